# baseline (speedup 1.0000x reference)
import jax
import jax.numpy as jnp
from jax import lax
from jax.experimental import pallas as pl
from jax.experimental.pallas import tpu as pltpu
from jax.experimental.pallas import tpu_sc as plsc

N_FEATS = 26
VOCAB = 100000
DIM = 64
BATCH = 16384

_NUM_WORKERS = 32
_RPC = 4
_CROWS = _RPC * N_FEATS
_NCHUNK_TOT = BATCH // _RPC
_C_PER_W = _NCHUNK_TOT // _NUM_WORKERS
_NBUF = 6
_WLAG = 2


def _encoder_body(w_hbm, gidx_hbm, out_hbm, idx_all, *scratch):
    bufs = scratch[:_NBUF]
    gsem = scratch[_NBUF:2 * _NBUF]
    wsem = scratch[2 * _NBUF:3 * _NBUF]
    wid = lax.axis_index("s") * 2 + lax.axis_index("c")
    cbase = wid * _C_PER_W
    pltpu.sync_copy(gidx_hbm.at[pl.ds(cbase, _C_PER_W)], idx_all)

    pend_g = [None] * _NBUF
    pend_w = [None] * _NBUF

    def start_gather(c):
        b = c % _NBUF
        pend_g[b] = pltpu.async_copy(
            w_hbm.at[idx_all.at[c]], bufs[b], gsem[b])

    def start_write(c):
        b = c % _NBUF
        pend_w[b] = pltpu.async_copy(
            bufs[b], out_hbm.at[pl.ds((cbase + c) * _CROWS, _CROWS)],
            wsem[b])

    for c in range(_NBUF):
        start_gather(c)
    for c in range(_C_PER_W):
        b = c % _NBUF
        pend_g[b].wait()
        start_write(c)
        d = c - _WLAG
        if d >= 0 and d + _NBUF < _C_PER_W:
            bd = d % _NBUF
            pend_w[bd].wait()
            pend_w[bd] = None
            start_gather(d + _NBUF)
    for p in pend_w:
        if p is not None:
            p.wait()


@jax.jit
def _encode(W, idxT):
    k = pl.kernel(
        _encoder_body,
        out_type=jax.ShapeDtypeStruct((BATCH * N_FEATS, DIM), jnp.float32),
        mesh=plsc.VectorSubcoreMesh(core_axis_name="c", subcore_axis_name="s"),
        scratch_types=(
            [pltpu.VMEM((_C_PER_W, _CROWS), jnp.int32)]
            + [pltpu.VMEM((_CROWS, DIM), jnp.float32) for _ in range(_NBUF)]
            + [pltpu.SemaphoreType.DMA for _ in range(2 * _NBUF)]
        ),
        compiler_params=pltpu.CompilerParams(use_tc_tiling_on_sc=False),
    )
    W_flat = W.reshape(N_FEATS * VOCAB, DIM)
    offs = (jnp.arange(N_FEATS, dtype=jnp.int32) * VOCAB)[None, :]
    gidx = (idxT + offs).reshape(_NCHUNK_TOT, _CROWS)
    out = k(W_flat, gidx)
    return out.reshape(BATCH, N_FEATS * DIM)


def kernel(W, cat_0, cat_1, cat_2, cat_3, cat_4, cat_5, cat_6, cat_7,
           cat_8, cat_9, cat_10, cat_11, cat_12, cat_13, cat_14, cat_15,
           cat_16, cat_17, cat_18, cat_19, cat_20, cat_21, cat_22, cat_23,
           cat_24, cat_25):
    cats = [cat_0, cat_1, cat_2, cat_3, cat_4, cat_5, cat_6, cat_7, cat_8,
            cat_9, cat_10, cat_11, cat_12, cat_13, cat_14, cat_15, cat_16,
            cat_17, cat_18, cat_19, cat_20, cat_21, cat_22, cat_23, cat_24,
            cat_25]
    idxT = jnp.stack(cats, axis=1)
    return _encode(W, idxT)
